# baseline (speedup 1.0000x reference)
import functools

import jax
import jax.numpy as jnp
from jax import lax
from jax.experimental import pallas as pl
from jax.experimental.pallas import tpu as pltpu
from jax.experimental.pallas import tpu_sc as plsc

N = 16384
K = 8192
D = 256
BETA = 0.5


BN = 1024
BK = 1024


def _argmin_body(esq_ref, z_ref, e_ref, idx_ref, lsum_ref, acc_s):
    z2 = -2.0 * z_ref[...]
    zsq = jnp.sum(z_ref[...] ** 2, axis=1, keepdims=True)
    lane = lax.broadcasted_iota(jnp.int32, (1, 128), 1).astype(jnp.float32)

    rmin = rcol = None
    for t in range(K // BK):
        s2 = lax.dot_general(
            z2, e_ref[t * BK:(t + 1) * BK, :],
            dimension_numbers=(((1,), (1,)), ((), ())),
            preferred_element_type=jnp.float32,
        )
        vals, cols = [], []
        for c in range(BK // 128):
            base = t * BK + c * 128
            vals.append((zsq + esq_ref[:, base:base + 128])
                        + s2[:, c * 128:(c + 1) * 128])
            cols.append(lane + float(base))
        while len(vals) > 1:
            nv, nc = [], []
            for p in range(0, len(vals), 2):
                better = vals[p + 1] < vals[p]
                nv.append(jnp.where(better, vals[p + 1], vals[p]))
                nc.append(jnp.where(better, cols[p + 1], cols[p]))
            vals, cols = nv, nc
        if rmin is None:
            rmin, rcol = vals[0], cols[0]
        else:
            better = vals[0] < rmin
            rmin = jnp.where(better, vals[0], rmin)
            rcol = jnp.where(better, cols[0], rcol)

    m = jnp.min(rmin, axis=1, keepdims=True)
    cand = jnp.where(rmin == m, rcol, jnp.float32(2 * K))
    idx_ref[...] = jnp.min(cand, axis=1, keepdims=True).astype(jnp.int32)

    i = pl.program_id(0)

    @pl.when(i == 0)
    def _linit():
        acc_s[...] = jnp.zeros_like(acc_s)

    acc_s[...] += jnp.sum(m, keepdims=True).reshape(1, 1)

    @pl.when(i == pl.num_programs(0) - 1)
    def _lflush():
        lsum_ref[...] = acc_s[...] * ((1.0 + BETA) / (N * D))


def _argmin_call(esq, z, e):
    return pl.pallas_call(
        _argmin_body,
        grid=(N // BN,),
        in_specs=[
            pl.BlockSpec((1, K), lambda i: (0, 0)),
            pl.BlockSpec((BN, D), lambda i: (i, 0)),
            pl.BlockSpec((K, D), lambda i: (0, 0)),
        ],
        out_specs=[
            pl.BlockSpec((BN, 1), lambda i: (i, 0)),
            pl.BlockSpec((1, 1), lambda i: (0, 0)),
        ],
        out_shape=[
            jax.ShapeDtypeStruct((N, 1), jnp.int32),
            jax.ShapeDtypeStruct((1, 1), jnp.float32),
        ],
        scratch_shapes=[pltpu.VMEM((1, 1), jnp.float32)],
        compiler_params=pltpu.CompilerParams(
            dimension_semantics=("arbitrary",),
            vmem_limit_bytes=100 * 1024 * 1024,
        ),
    )(esq, z, e)


_SC_CORES = 2
_SC_SUBCORES = 16
_NW = _SC_CORES * _SC_SUBCORES
_ROWS_PER_W = N // _NW
_CHUNK = 128
_NCHUNK = _ROWS_PER_W // _CHUNK


def _gather_call(table, idx):
    mesh = plsc.VectorSubcoreMesh(
        core_axis_name="c", subcore_axis_name="s",
        num_cores=_SC_CORES, num_subcores=_SC_SUBCORES,
    )

    @functools.partial(
        pl.kernel,
        out_type=jax.ShapeDtypeStruct((N, D), jnp.float32),
        mesh=mesh,
        scratch_types=[
            pltpu.VMEM((_CHUNK,), jnp.int32),
            pltpu.VMEM((_CHUNK, D), jnp.float32),
            pltpu.SemaphoreType.DMA,
        ],
    )
    def gather_kernel(table_hbm, idx_hbm, out_hbm, idx_v, rows_v, sem):
        wid = lax.axis_index("s") * _SC_CORES + lax.axis_index("c")
        base = wid * _ROWS_PER_W
        for c in range(_NCHUNK):
            off = base + c * _CHUNK
            pltpu.sync_copy(idx_hbm.at[pl.ds(off, _CHUNK)], idx_v)
            pltpu.async_copy(table_hbm.at[idx_v], rows_v, sem).wait()
            pltpu.sync_copy(rows_v, out_hbm.at[pl.ds(off, _CHUNK)])

    return gather_kernel(table, idx)


BE = 2048


def _st_body(z_ref, zq_ref, out_ref):
    z = z_ref[...]
    out_ref[...] = z + (zq_ref[...] - z)


def _st_call(z, zq):
    return pl.pallas_call(
        _st_body,
        grid=(N // BE,),
        in_specs=[
            pl.BlockSpec((BE, D), lambda i: (i, 0)),
            pl.BlockSpec((BE, D), lambda i: (i, 0)),
        ],
        out_specs=pl.BlockSpec((BE, D), lambda i: (i, 0)),
        out_shape=jax.ShapeDtypeStruct((N, D), jnp.float32),
        compiler_params=pltpu.CompilerParams(
            dimension_semantics=("arbitrary",),
        ),
    )(z, zq)


def kernel(z, embedding_weight):
    esq = jnp.sum(embedding_weight ** 2, axis=1)
    idx2, loss11 = _argmin_call(esq.reshape(1, K), z, embedding_weight)
    indices = idx2.reshape(N)
    zq = _gather_call(embedding_weight, indices)
    zq_out = _st_call(z, zq)
    return zq_out, indices, loss11.reshape(())


# speedup vs baseline: 1.0083x; 1.0083x over previous
import functools

import jax
import jax.numpy as jnp
from jax import lax
from jax.experimental import pallas as pl
from jax.experimental.pallas import tpu as pltpu
from jax.experimental.pallas import tpu_sc as plsc

N = 16384
K = 8192
D = 256
BETA = 0.5


BN = 1024
BK = 2048


def _argmin_body(esq_ref, z_ref, e_ref, idx_ref):
    z2 = -2.0 * z_ref[...]
    zsq = jnp.sum(z_ref[...] ** 2, axis=1, keepdims=True)
    lane = lax.broadcasted_iota(jnp.int32, (1, 128), 1).astype(jnp.float32)

    rmin = rcol = None
    for t in range(K // BK):
        s2 = lax.dot_general(
            z2, e_ref[t * BK:(t + 1) * BK, :],
            dimension_numbers=(((1,), (1,)), ((), ())),
            preferred_element_type=jnp.float32,
        )
        vals, cols = [], []
        for c in range(BK // 128):
            base = t * BK + c * 128
            vals.append((zsq + esq_ref[:, base:base + 128])
                        + s2[:, c * 128:(c + 1) * 128])
            cols.append(lane + float(base))
        while len(vals) > 1:
            nv, nc = [], []
            for p in range(0, len(vals), 2):
                better = vals[p + 1] < vals[p]
                nv.append(jnp.where(better, vals[p + 1], vals[p]))
                nc.append(jnp.where(better, cols[p + 1], cols[p]))
            vals, cols = nv, nc
        if rmin is None:
            rmin, rcol = vals[0], cols[0]
        else:
            better = vals[0] < rmin
            rmin = jnp.where(better, vals[0], rmin)
            rcol = jnp.where(better, cols[0], rcol)

    m = jnp.min(rmin, axis=1, keepdims=True)
    cand = jnp.where(rmin == m, rcol, jnp.float32(2 * K))
    idx_ref[...] = jnp.min(cand, axis=1, keepdims=True).astype(jnp.int32)


def _argmin_call(esq, z, e):
    return pl.pallas_call(
        _argmin_body,
        grid=(N // BN,),
        in_specs=[
            pl.BlockSpec((1, K), lambda i: (0, 0)),
            pl.BlockSpec((BN, D), lambda i: (i, 0)),
            pl.BlockSpec((K, D), lambda i: (0, 0)),
        ],
        out_specs=pl.BlockSpec((BN, 1), lambda i: (i, 0)),
        out_shape=jax.ShapeDtypeStruct((N, 1), jnp.int32),
        compiler_params=pltpu.CompilerParams(
            dimension_semantics=("arbitrary",),
            vmem_limit_bytes=100 * 1024 * 1024,
        ),
    )(esq, z, e)


_SC_CORES = 2
_SC_SUBCORES = 16
_NW = _SC_CORES * _SC_SUBCORES
_ROWS_PER_W = N // _NW
_CHUNK = 128
_NCHUNK = _ROWS_PER_W // _CHUNK


def _gather_call(table, idx):
    mesh = plsc.VectorSubcoreMesh(
        core_axis_name="c", subcore_axis_name="s",
        num_cores=_SC_CORES, num_subcores=_SC_SUBCORES,
    )

    @functools.partial(
        pl.kernel,
        out_type=jax.ShapeDtypeStruct((N, D), jnp.float32),
        mesh=mesh,
        scratch_types=[
            pltpu.VMEM((_CHUNK,), jnp.int32),
            pltpu.VMEM((_CHUNK, D), jnp.float32),
            pltpu.SemaphoreType.DMA,
        ],
    )
    def gather_kernel(table_hbm, idx_hbm, out_hbm, idx_v, rows_v, sem):
        wid = lax.axis_index("s") * _SC_CORES + lax.axis_index("c")
        base = wid * _ROWS_PER_W
        for c in range(_NCHUNK):
            off = base + c * _CHUNK
            pltpu.sync_copy(idx_hbm.at[pl.ds(off, _CHUNK)], idx_v)
            pltpu.async_copy(table_hbm.at[idx_v], rows_v, sem).wait()
            pltpu.sync_copy(rows_v, out_hbm.at[pl.ds(off, _CHUNK)])

    return gather_kernel(table, idx)


BE = 2048


def _st_loss_body(z_ref, zq_ref, out_ref, loss_ref, acc_s):
    i = pl.program_id(0)
    ni = pl.num_programs(0)
    zq = zq_ref[...]
    z = z_ref[...]
    diff = zq - z
    out_ref[...] = z + diff

    @pl.when(i == 0)
    def _init():
        acc_s[...] = jnp.zeros_like(acc_s)

    acc_s[...] += jnp.sum(diff * diff, keepdims=True).reshape(1, 1)

    @pl.when(i == ni - 1)
    def _flush():
        loss_ref[...] = acc_s[...] * ((1.0 + BETA) / (N * D))


def _st_loss_call(z, zq):
    return pl.pallas_call(
        _st_loss_body,
        grid=(N // BE,),
        in_specs=[
            pl.BlockSpec((BE, D), lambda i: (i, 0)),
            pl.BlockSpec((BE, D), lambda i: (i, 0)),
        ],
        out_specs=[
            pl.BlockSpec((BE, D), lambda i: (i, 0)),
            pl.BlockSpec((1, 1), lambda i: (0, 0)),
        ],
        out_shape=[
            jax.ShapeDtypeStruct((N, D), jnp.float32),
            jax.ShapeDtypeStruct((1, 1), jnp.float32),
        ],
        scratch_shapes=[pltpu.VMEM((1, 1), jnp.float32)],
        compiler_params=pltpu.CompilerParams(
            dimension_semantics=("arbitrary",),
        ),
    )(z, zq)


def kernel(z, embedding_weight):
    esq = jnp.sum(embedding_weight ** 2, axis=1)
    idx2 = _argmin_call(esq.reshape(1, K), z, embedding_weight)
    indices = idx2.reshape(N)
    zq = _gather_call(embedding_weight, indices)
    zq_out, loss11 = _st_loss_call(z, zq)
    return zq_out, indices, loss11.reshape(())


# speedup vs baseline: 1.0235x; 1.0151x over previous
import functools

import jax
import jax.numpy as jnp
from jax import lax
from jax.experimental import pallas as pl
from jax.experimental.pallas import tpu as pltpu
from jax.experimental.pallas import tpu_sc as plsc

N = 16384
K = 8192
D = 256
BETA = 0.5


BN = 1024
BK = 1024


def _argmin_body(esq_ref, z_ref, e_ref, idx_ref):
    z2 = -2.0 * z_ref[...]
    zsq = jnp.sum(z_ref[...] ** 2, axis=1, keepdims=True)
    lane = lax.broadcasted_iota(jnp.int32, (1, 128), 1).astype(jnp.float32)

    rmin = rcol = None
    for t in range(K // BK):
        s2 = lax.dot_general(
            z2, e_ref[t * BK:(t + 1) * BK, :],
            dimension_numbers=(((1,), (1,)), ((), ())),
            preferred_element_type=jnp.float32,
        )
        vals, cols = [], []
        for c in range(BK // 128):
            base = t * BK + c * 128
            vals.append((zsq + esq_ref[:, base:base + 128])
                        + s2[:, c * 128:(c + 1) * 128])
            cols.append(lane + float(base))
        while len(vals) > 1:
            nv, nc = [], []
            for p in range(0, len(vals), 2):
                better = vals[p + 1] < vals[p]
                nv.append(jnp.where(better, vals[p + 1], vals[p]))
                nc.append(jnp.where(better, cols[p + 1], cols[p]))
            vals, cols = nv, nc
        if rmin is None:
            rmin, rcol = vals[0], cols[0]
        else:
            better = vals[0] < rmin
            rmin = jnp.where(better, vals[0], rmin)
            rcol = jnp.where(better, cols[0], rcol)

    m = jnp.min(rmin, axis=1, keepdims=True)
    cand = jnp.where(rmin == m, rcol, jnp.float32(2 * K))
    idx_ref[...] = jnp.min(cand, axis=1, keepdims=True).astype(jnp.int32)


def _argmin_call(esq, z, e):
    return pl.pallas_call(
        _argmin_body,
        grid=(N // BN,),
        in_specs=[
            pl.BlockSpec((1, K), lambda i: (0, 0)),
            pl.BlockSpec((BN, D), lambda i: (i, 0)),
            pl.BlockSpec((K, D), lambda i: (0, 0)),
        ],
        out_specs=pl.BlockSpec((BN, 1), lambda i: (i, 0)),
        out_shape=jax.ShapeDtypeStruct((N, 1), jnp.int32),
        compiler_params=pltpu.CompilerParams(
            dimension_semantics=("arbitrary",),
            vmem_limit_bytes=100 * 1024 * 1024,
        ),
    )(esq, z, e)


_SC_CORES = 2
_SC_SUBCORES = 16
_NW = _SC_CORES * _SC_SUBCORES
_ROWS_PER_W = N // _NW
_CHUNK = 128
_NCHUNK = _ROWS_PER_W // _CHUNK


def _gather_call(table, idx):
    mesh = plsc.VectorSubcoreMesh(
        core_axis_name="c", subcore_axis_name="s",
        num_cores=_SC_CORES, num_subcores=_SC_SUBCORES,
    )

    @functools.partial(
        pl.kernel,
        out_type=jax.ShapeDtypeStruct((N, D), jnp.float32),
        mesh=mesh,
        scratch_types=[
            pltpu.VMEM((_NCHUNK, _CHUNK), jnp.int32),
            pltpu.VMEM((_CHUNK, D), jnp.float32),
            pltpu.VMEM((_CHUNK, D), jnp.float32),
            pltpu.SemaphoreType.DMA,
            pltpu.SemaphoreType.DMA,
        ],
    )
    def gather_kernel(table_hbm, idx_hbm, out_hbm, idx_v, r0, r1, s0, s1):
        wid = lax.axis_index("s") * _SC_CORES + lax.axis_index("c")
        base = wid * _ROWS_PER_W
        pltpu.sync_copy(idx_hbm.at[pl.ds(wid * _NCHUNK, _NCHUNK)], idx_v)
        bufs, sems = (r0, r1), (s0, s1)
        handles = [None] * _NCHUNK
        for c in range(2):
            handles[c] = pltpu.async_copy(
                table_hbm.at[idx_v.at[c]], bufs[c % 2], sems[c % 2])
        for c in range(_NCHUNK):
            handles[c].wait()
            pltpu.sync_copy(bufs[c % 2],
                            out_hbm.at[pl.ds(base + c * _CHUNK, _CHUNK)])
            if c + 2 < _NCHUNK:
                handles[c + 2] = pltpu.async_copy(
                    table_hbm.at[idx_v.at[c + 2]],
                    bufs[(c + 2) % 2], sems[(c + 2) % 2])

    return gather_kernel(table, idx.reshape(N // _CHUNK, _CHUNK))


BE = 2048


def _st_loss_body(z_ref, zq_ref, out_ref, loss_ref, acc_s):
    i = pl.program_id(0)
    ni = pl.num_programs(0)
    zq = zq_ref[...]
    z = z_ref[...]
    diff = zq - z
    out_ref[...] = z + diff

    @pl.when(i == 0)
    def _init():
        acc_s[...] = jnp.zeros_like(acc_s)

    acc_s[...] += jnp.sum(diff * diff, keepdims=True).reshape(1, 1)

    @pl.when(i == ni - 1)
    def _flush():
        loss_ref[...] = acc_s[...] * ((1.0 + BETA) / (N * D))


def _st_loss_call(z, zq):
    return pl.pallas_call(
        _st_loss_body,
        grid=(N // BE,),
        in_specs=[
            pl.BlockSpec((BE, D), lambda i: (i, 0)),
            pl.BlockSpec((BE, D), lambda i: (i, 0)),
        ],
        out_specs=[
            pl.BlockSpec((BE, D), lambda i: (i, 0)),
            pl.BlockSpec((1, 1), lambda i: (0, 0)),
        ],
        out_shape=[
            jax.ShapeDtypeStruct((N, D), jnp.float32),
            jax.ShapeDtypeStruct((1, 1), jnp.float32),
        ],
        scratch_shapes=[pltpu.VMEM((1, 1), jnp.float32)],
        compiler_params=pltpu.CompilerParams(
            dimension_semantics=("arbitrary",),
        ),
    )(z, zq)


def kernel(z, embedding_weight):
    esq = jnp.sum(embedding_weight ** 2, axis=1)
    idx2 = _argmin_call(esq.reshape(1, K), z, embedding_weight)
    indices = idx2.reshape(N)
    zq = _gather_call(embedding_weight, indices)
    zq_out, loss11 = _st_loss_call(z, zq)
    return zq_out, indices, loss11.reshape(())
